# baseline (speedup 1.0000x reference)
import functools

import jax
import jax.numpy as jnp
from jax import lax
from jax.experimental import pallas as pl
from jax.experimental.pallas import tpu as pltpu
from jax.experimental.pallas import tpu_sc as plsc

_N_USERS = 25000
_N_ITEMS = 25000
_N = _N_USERS + _N_ITEMS
_D = 64
_H = 32
_E = 800000

_NC = 2
_NS = 16
_CH = 128
_KB = 3
_BB = _CH * _KB
_NB = 131
_EPT = _BB * _NB
_EPAD = _EPT * _NS

_SLAB = 3128
_SLAB_LAST = _N - 15 * _SLAB


def _spmm_body(x_ref, z_ref, src_ref, dst_ref, w_ref, y_ref,
               sidx0, sidx1, wbuf0, wbuf1, didx0, didx1, didx2,
               rows0, rows1, acc, esem, gsem, ssem):
  c = lax.axis_index("c")
  s = lax.axis_index("s")

  @pl.when(s < 15)
  def _():
    pltpu.sync_copy(z_ref, acc.at[pl.ds(s * _SLAB, _SLAB)])

  @pl.when(s == 15)
  def _():
    pltpu.sync_copy(z_ref.at[pl.ds(0, _SLAB_LAST)],
                    acc.at[pl.ds(15 * _SLAB, _SLAB_LAST)])

  plsc.subcore_barrier()

  sidx = (sidx0, sidx1)
  wbufs = (wbuf0, wbuf1)
  didx = (didx0, didx1, didx2)

  def edge_fire(j, p, t):
    pltpu.async_copy(src_ref.at[s, j], sidx[p], esem)
    pltpu.async_copy(dst_ref.at[s, j], didx[t], esem)
    pltpu.async_copy(w_ref.at[s, j], wbufs[p], esem)

  def edge_wait():
    pltpu.make_async_copy(src_ref.at[s, 0], sidx[0], esem).wait()
    pltpu.make_async_copy(dst_ref.at[s, 0], didx[0], esem).wait()
    pltpu.make_async_copy(w_ref.at[s, 0], wbufs[0], esem).wait()

  coff = c * _N

  def offset_src(p):
    for k in range(_KB):
      for q in range(_CH // 16):
        v = sidx[p][k, pl.ds(q * 16, 16)]
        sidx[p][k, pl.ds(q * 16, 16)] = v + coff

  def gathers_fire(p):
    for k in range(_KB):
      pltpu.async_copy(x_ref.at[sidx[p].at[k]],
                       rows0.at[pl.ds(k * _CH, _CH)] if p == 0
                       else rows1.at[pl.ds(k * _CH, _CH)], gsem)

  def gathers_wait(p):
    for k in range(_KB):
      pltpu.make_async_copy(x_ref.at[sidx[p].at[k]],
                            rows0.at[pl.ds(k * _CH, _CH)] if p == 0
                            else rows1.at[pl.ds(k * _CH, _CH)], gsem).wait()

  def scatters_fire(p, t):
    rows = rows0 if p == 0 else rows1
    for k in range(_KB):
      pltpu.async_copy(rows.at[pl.ds(k * _CH, _CH)],
                       acc.at[didx[t].at[k]], ssem, add=True)

  def scatters_wait():
    for k in range(_KB):
      pltpu.make_async_copy(rows0.at[pl.ds(k * _CH, _CH)],
                            acc.at[didx[0].at[k]], ssem).wait()

  splat_dnums = lax.GatherDimensionNumbers(
      offset_dims=(), collapsed_slice_dims=(0,), start_index_map=(0,))
  splat_idx = [jnp.full((16, 1), u, jnp.int32) for u in range(16)]

  def mul(p):
    rows = rows0 if p == 0 else rows1
    wb = wbufs[p]

    def _mul(m, _):
      w16 = wb[pl.ds(m * 16, 16)]
      for u in range(16):
        e = m * 16 + u
        w = lax.gather(w16, splat_idx[u], splat_dnums, slice_sizes=(1,),
                       mode=lax.GatherScatterMode.PROMISE_IN_BOUNDS)
        rows[e, pl.ds(0, 16)] = rows[e, pl.ds(0, 16)] * w
        rows[e, pl.ds(16, 16)] = rows[e, pl.ds(16, 16)] * w
      return _

    lax.fori_loop(0, _BB // 16, _mul, None)

  edge_fire(0, 0, 0)
  edge_wait()
  offset_src(0)
  gathers_fire(0)
  edge_fire(1, 1, 1)

  def step(j, p, t):
    gathers_wait(p)

    @pl.when(j >= 1)
    def _():
      scatters_wait()

    @pl.when(j + 1 < _NB)
    def _():
      edge_wait()
      offset_src(1 - p)
      gathers_fire(1 - p)

    mul(p)

    @pl.when(j + 2 < _NB)
    def _():
      if t == 0:
        edge_fire(j + 2, p, 2)
      elif t == 1:
        edge_fire(j + 2, p, 0)
      else:
        edge_fire(j + 2, p, 1)

    scatters_fire(p, t)

  def body(j, _):
    for r in range(6):
      @pl.when(j % 6 == r)
      def _(r=r):
        step(j, r % 2, r % 3)
    return _

  lax.fori_loop(0, _NB, body, None)
  scatters_wait()
  plsc.subcore_barrier()

  @pl.when(s < 15)
  def _():
    pltpu.sync_copy(acc.at[pl.ds(s * _SLAB, _SLAB)],
                    y_ref.at[pl.ds(c * _N + s * _SLAB, _SLAB)])

  @pl.when(s == 15)
  def _():
    pltpu.sync_copy(acc.at[pl.ds(15 * _SLAB, _SLAB_LAST)],
                    y_ref.at[pl.ds(c * _N + 15 * _SLAB, _SLAB_LAST)])


@functools.lru_cache(maxsize=None)
def _make_spmm():
  return pl.kernel(
    _spmm_body,
    out_type=jax.ShapeDtypeStruct((_NC * _N, _H), jnp.float32),
    mesh=plsc.VectorSubcoreMesh(core_axis_name="c", subcore_axis_name="s",
                                num_cores=_NC, num_subcores=_NS),
    compiler_params=pltpu.CompilerParams(use_tc_tiling_on_sc=False),
    scratch_types=[
        pltpu.VMEM((_KB, _CH), jnp.int32),
        pltpu.VMEM((_KB, _CH), jnp.int32),
        pltpu.VMEM((_BB,), jnp.float32),
        pltpu.VMEM((_BB,), jnp.float32),
        pltpu.VMEM((_KB, _CH), jnp.int32),
        pltpu.VMEM((_KB, _CH), jnp.int32),
        pltpu.VMEM((_KB, _CH), jnp.int32),
        pltpu.VMEM((_BB, _H), jnp.float32),
        pltpu.VMEM((_BB, _H), jnp.float32),
        pltpu.VMEM_SHARED((_N, _H), jnp.float32),
        pltpu.SemaphoreType.DMA,
        pltpu.SemaphoreType.DMA,
        pltpu.SemaphoreType.DMA,
    ],
  )


def _prep_edges(edge_index, edge_weight):
  pad = _EPAD - _E
  src4 = jnp.pad(edge_index[1], (0, pad)).reshape(_NS, _NB, _KB, _CH)
  dst4 = jnp.pad(edge_index[0], (0, pad)).reshape(_NS, _NB, _KB, _CH)
  w3 = jnp.pad(edge_weight, (0, pad)).reshape(_NS, _NB, _BB)
  return src4, dst4, w3


_FCH = 184


def _fuse_body_factory(scale):
  def _fuse_body(yb_ref, yc_ref, g_ref, acc_ref, xn_ref, ao_ref,
                 byb0, byc0, bg0, bacc0, bxn0, bao0,
                 byb1, byc1, bg1, bacc1, bxn1, bao1, isem, osem):
    c = lax.axis_index("c")
    s = lax.axis_index("s")
    row0 = c * _N + s * _SLAB
    g0 = s * _SLAB

    ins = ((byb0, byc0, bg0, bacc0), (byb1, byc1, bg1, bacc1))
    outs = ((bxn0, bao0), (bxn1, bao1))

    def in_fire(j, p):
      byb, byc, bg, bacc = ins[p]
      pltpu.async_copy(yb_ref.at[pl.ds(row0 + j * _FCH, _FCH)], byb, isem)
      pltpu.async_copy(yc_ref.at[pl.ds(row0 + j * _FCH, _FCH)], byc, isem)
      pltpu.async_copy(g_ref.at[pl.ds(g0 + j * _FCH, _FCH)], bg, isem)
      pltpu.async_copy(acc_ref.at[pl.ds(row0 + j * _FCH, _FCH)], bacc, isem)

    def in_wait(p):
      byb, byc, bg, bacc = ins[p]
      pltpu.make_async_copy(yb_ref.at[pl.ds(row0, _FCH)], byb, isem).wait()
      pltpu.make_async_copy(yc_ref.at[pl.ds(row0, _FCH)], byc, isem).wait()
      pltpu.make_async_copy(g_ref.at[pl.ds(g0, _FCH)], bg, isem).wait()
      pltpu.make_async_copy(acc_ref.at[pl.ds(row0, _FCH)], bacc, isem).wait()

    def out_fire(j, p):
      bxn, bao = outs[p]
      pltpu.async_copy(bxn, xn_ref.at[pl.ds(row0 + j * _FCH, _FCH)], osem)
      pltpu.async_copy(bao, ao_ref.at[pl.ds(row0 + j * _FCH, _FCH)], osem)

    def out_wait(p):
      bxn, bao = outs[p]
      pltpu.make_async_copy(bxn, xn_ref.at[pl.ds(row0, _FCH)], osem).wait()
      pltpu.make_async_copy(bao, ao_ref.at[pl.ds(row0, _FCH)], osem).wait()

    def compute(p, nrows):
      byb, byc, bg, bacc = ins[p]
      bxn, bao = outs[p]

      def rowfn(i, _):
        g = bg[i, pl.ds(0, 16)]
        bl = byb[i, pl.ds(0, 16)]
        bh = byb[i, pl.ds(16, 16)]
        cl = byc[i, pl.ds(0, 16)]
        ch = byc[i, pl.ds(16, 16)]
        nl = g * (bl - cl) + cl
        nh = g * (bh - ch) + ch
        bxn[i, pl.ds(0, 16)] = nl
        bxn[i, pl.ds(16, 16)] = nh
        al = bacc[i, pl.ds(0, 16)] + nl
        ah = bacc[i, pl.ds(16, 16)] + nh
        if scale != 1.0:
          al = al * scale
          ah = ah * scale
        bao[i, pl.ds(0, 16)] = al
        bao[i, pl.ds(16, 16)] = ah
        return _

      lax.fori_loop(0, nrows, rowfn, None)

    nch = _SLAB // _FCH
    nch_last = _SLAB_LAST // _FCH

    def run(nfull):
      in_fire(0, 0)

      def chunk(j, _):
        @pl.when(j % 2 == 0)
        def _():
          stepc(j, 0)

        @pl.when(j % 2 == 1)
        def _():
          stepc(j, 1)
        return _

      def stepc(j, p):
        in_wait(p)

        @pl.when(j + 1 < nfull)
        def _():
          in_fire(j + 1, 1 - p)

        @pl.when(j >= 2)
        def _():
          out_wait(p)

        compute(p, _FCH)
        out_fire(j, p)

      lax.fori_loop(0, nfull, chunk, None)
      if nfull >= 2:
        out_wait(nfull % 2)
      out_wait((nfull + 1) % 2)

    @pl.when(s < 15)
    def _():
      run(nch)

    @pl.when(s == 15)
    def _():
      run(nch_last)
      t0 = row0 + nch_last * _FCH
      gt0 = g0 + nch_last * _FCH
      byb, byc, bg, bacc = ins[0]
      bxn, bao = outs[0]
      pltpu.sync_copy(yb_ref.at[pl.ds(t0, 136)], byb.at[pl.ds(0, 136)])
      pltpu.sync_copy(yc_ref.at[pl.ds(t0, 136)], byc.at[pl.ds(0, 136)])
      pltpu.sync_copy(g_ref.at[pl.ds(gt0, 136)], bg.at[pl.ds(0, 136)])
      pltpu.sync_copy(acc_ref.at[pl.ds(t0, 136)], bacc.at[pl.ds(0, 136)])
      compute(0, 136)
      pltpu.sync_copy(bxn.at[pl.ds(0, 136)], xn_ref.at[pl.ds(t0, 136)])
      pltpu.sync_copy(bao.at[pl.ds(0, 136)], ao_ref.at[pl.ds(t0, 136)])

  return _fuse_body


@functools.lru_cache(maxsize=None)
def _make_fuse(scale):
  buf = lambda w: pltpu.VMEM((_FCH, w), jnp.float32)
  return pl.kernel(
      _fuse_body_factory(scale),
      out_type=(jax.ShapeDtypeStruct((_NC * _N, _H), jnp.float32),
                jax.ShapeDtypeStruct((_NC * _N, _H), jnp.float32)),
      mesh=plsc.VectorSubcoreMesh(core_axis_name="c", subcore_axis_name="s",
                                  num_cores=_NC, num_subcores=_NS),
      compiler_params=pltpu.CompilerParams(use_tc_tiling_on_sc=False),
      scratch_types=[
          buf(32), buf(32), buf(_H), buf(32), buf(32), buf(32),
          buf(32), buf(32), buf(_H), buf(32), buf(32), buf(32),
          pltpu.SemaphoreType.DMA,
          pltpu.SemaphoreType.DMA,
      ],
  )


_BLK = 1000
_GRID = _N // _BLK
_UBLKS = _N_USERS // _BLK


def _gate(gi, w1, b1, w2, b2):
  h = lax.dot_general(gi, w1, (((1,), (1,)), ((), ())),
                      preferred_element_type=jnp.float32)
  h = jnp.maximum(h + b1, 0.0)
  g = lax.dot_general(h, w2, (((1,), (1,)), ((), ())),
                      preferred_element_type=jnp.float32)
  return jax.nn.sigmoid(g + b2)


def _rep_w2(w2, b2, w=_H):
  return jnp.tile(w2, (w, 1)), jnp.tile(b2.reshape(1, 1), (1, w))


def _init_body(x1_ref, x2_ref, w1_ref, b1_ref, w2_ref, b2_ref, o_ref):
  x1 = x1_ref[...]
  x2 = x2_ref[...]
  g = _gate(jnp.concatenate([x1, x2], axis=1),
            w1_ref[...], b1_ref[...], w2_ref[...], b2_ref[...])
  out_lo = g * x1[:, :_H] + (1.0 - g) * x2[:, :_H]
  out_hi = g * x1[:, _H:] + (1.0 - g) * x2[:, _H:]
  o_ref[0] = out_lo
  o_ref[1] = out_hi


def _init_fuse(a, b, w1, b1, w2, b2):
  return pl.pallas_call(
      _init_body,
      grid=(_GRID,),
      in_specs=[
          pl.BlockSpec((_BLK, _D), lambda i: (i, 0)),
          pl.BlockSpec((_BLK, _D), lambda i: (i, 0)),
          pl.BlockSpec((32, 2 * _D), lambda i: (0, 0)),
          pl.BlockSpec((1, 32), lambda i: (0, 0)),
          pl.BlockSpec((_H, 32), lambda i: (0, 0)),
          pl.BlockSpec((1, _H), lambda i: (0, 0)),
      ],
      out_specs=pl.BlockSpec((2, _BLK, _H), lambda i: (0, i, 0)),
      out_shape=jax.ShapeDtypeStruct((2, _N, _H), jnp.float32),
  )(a, b, w1, b1, w2, b2)


def _gate_body(yb_ref, herb_ref, w1_ref, b1_ref, w2_ref, b2_ref, g_ref):
  g_ref[...] = _gate(
      jnp.concatenate([yb_ref[0], yb_ref[1], herb_ref[...]], axis=1),
      w1_ref[...], b1_ref[...], w2_ref[...], b2_ref[...])


def _item_gates(yb3, herb, w1, b1, w2, b2):
  return pl.pallas_call(
      _gate_body,
      grid=(_N_ITEMS // _BLK,),
      in_specs=[
          pl.BlockSpec((2, _BLK, _H), lambda i: (0, _UBLKS + i, 0)),
          pl.BlockSpec((_BLK, _D), lambda i: (i, 0)),
          pl.BlockSpec((32, 2 * _D), lambda i: (0, 0)),
          pl.BlockSpec((1, 32), lambda i: (0, 0)),
          pl.BlockSpec((_H, 32), lambda i: (0, 0)),
          pl.BlockSpec((1, _H), lambda i: (0, 0)),
      ],
      out_specs=pl.BlockSpec((_BLK, _H), lambda i: (i, 0)),
      out_shape=jax.ShapeDtypeStruct((_N_ITEMS, _H), jnp.float32),
  )(yb3, herb, w1, b1, w2, b2)


def kernel(users_emb, items_emb, symptom_emb, herb_emb, base_edge_index,
           base_edge_weight, cooccur_edge_index, cooccur_edge_weight,
           gate_W1, gate_b1, gate_W2, gate_b2):
  n_layers = gate_W1.shape[0] - 1

  a = jnp.concatenate([users_emb, items_emb], axis=0)
  b = jnp.concatenate([symptom_emb, herb_emb], axis=0)
  w2r, b2r = _rep_w2(gate_W2[0], gate_b2[0])
  all_emb = _init_fuse(a, b, gate_W1[0], gate_b1[0:1], w2r, b2r)

  base_edges = _prep_edges(base_edge_index, base_edge_weight)
  cooc_edges = _prep_edges(cooccur_edge_index, cooccur_edge_weight)

  spmm = _make_spmm()
  zrows = jnp.zeros((_SLAB, _H), jnp.float32)
  ones_g = jnp.ones((_N_USERS, _H), jnp.float32)
  xcur = all_emb.reshape(_NC * _N, _H)
  acc = xcur
  for layer in range(1, n_layers + 1):
    ybase = spmm(xcur, zrows, *base_edges)
    ycooc = spmm(xcur, zrows, *cooc_edges)
    w2r, b2r = _rep_w2(gate_W2[layer], gate_b2[layer])
    g_items = _item_gates(ybase.reshape(2, _N, _H), herb_emb,
                          gate_W1[layer], gate_b1[layer:layer + 1], w2r, b2r)
    gfull = jnp.concatenate([ones_g, g_items], axis=0)
    scale = 1.0 / (n_layers + 1) if layer == n_layers else 1.0
    xcur, acc = _make_fuse(scale)(ybase, ycooc, gfull, acc)

  users = jnp.concatenate([acc[:_N_USERS], acc[_N: _N + _N_USERS]], axis=1)
  items = jnp.concatenate([acc[_N_USERS:_N], acc[_N + _N_USERS:]], axis=1)
  return (users, items)


# speedup vs baseline: 1.0012x; 1.0012x over previous
import functools

import jax
import jax.numpy as jnp
from jax import lax
from jax.experimental import pallas as pl
from jax.experimental.pallas import tpu as pltpu
from jax.experimental.pallas import tpu_sc as plsc

_N_USERS = 25000
_N_ITEMS = 25000
_N = _N_USERS + _N_ITEMS
_D = 64
_H = 32
_E = 800000

_NC = 2
_NS = 16
_CH = 128
_KB = 3
_BB = _CH * _KB
_NB = 131
_EPT = _BB * _NB
_EPAD = _EPT * _NS

_SLAB = 3128
_SLAB_LAST = _N - 15 * _SLAB


def _spmm_body(x_ref, z_ref, src_ref, dst_ref, w_ref, y_ref,
               sidx0, sidx1, wbuf0, wbuf1, didx0, didx1, didx2,
               rows0, rows1, acc, esem, gsem, ssem):
  c = lax.axis_index("c")
  s = lax.axis_index("s")

  @pl.when(s < 15)
  def _():
    pltpu.sync_copy(z_ref, acc.at[pl.ds(s * _SLAB, _SLAB)])

  @pl.when(s == 15)
  def _():
    pltpu.sync_copy(z_ref.at[pl.ds(0, _SLAB_LAST)],
                    acc.at[pl.ds(15 * _SLAB, _SLAB_LAST)])

  plsc.subcore_barrier()

  sidx = (sidx0, sidx1)
  wbufs = (wbuf0, wbuf1)
  didx = (didx0, didx1, didx2)

  def edge_fire(j, p, t):
    pltpu.async_copy(src_ref.at[s, j], sidx[p], esem)
    pltpu.async_copy(dst_ref.at[s, j], didx[t], esem)
    pltpu.async_copy(w_ref.at[s, j], wbufs[p], esem)

  def edge_wait():
    pltpu.make_async_copy(src_ref.at[s, 0], sidx[0], esem).wait()
    pltpu.make_async_copy(dst_ref.at[s, 0], didx[0], esem).wait()
    pltpu.make_async_copy(w_ref.at[s, 0], wbufs[0], esem).wait()

  coff = c * _N

  def offset_src(p):
    for k in range(_KB):
      for q in range(_CH // 16):
        v = sidx[p][k, pl.ds(q * 16, 16)]
        sidx[p][k, pl.ds(q * 16, 16)] = v + coff

  def gathers_fire(p):
    for k in range(_KB):
      pltpu.async_copy(x_ref.at[sidx[p].at[k]],
                       rows0.at[pl.ds(k * _CH, _CH)] if p == 0
                       else rows1.at[pl.ds(k * _CH, _CH)], gsem)

  def gathers_wait(p):
    for k in range(_KB):
      pltpu.make_async_copy(x_ref.at[sidx[p].at[k]],
                            rows0.at[pl.ds(k * _CH, _CH)] if p == 0
                            else rows1.at[pl.ds(k * _CH, _CH)], gsem).wait()

  def scatters_fire(p, t):
    rows = rows0 if p == 0 else rows1
    for k in range(_KB):
      pltpu.async_copy(rows.at[pl.ds(k * _CH, _CH)],
                       acc.at[didx[t].at[k]], ssem, add=True)

  def scatters_wait():
    for k in range(_KB):
      pltpu.make_async_copy(rows0.at[pl.ds(k * _CH, _CH)],
                            acc.at[didx[0].at[k]], ssem).wait()

  def mul(p):
    rows = rows0 if p == 0 else rows1
    wb = wbufs[p]

    def _mul(m, _):
      w16 = wb[pl.ds(m * 16, 16)]
      for u in range(16):
        e = m * 16 + u
        w = w16[u]
        rows[e, pl.ds(0, 16)] = rows[e, pl.ds(0, 16)] * w
        rows[e, pl.ds(16, 16)] = rows[e, pl.ds(16, 16)] * w
      return _

    lax.fori_loop(0, _BB // 16, _mul, None)

  edge_fire(0, 0, 0)
  edge_wait()
  offset_src(0)
  gathers_fire(0)
  edge_fire(1, 1, 1)

  def step(j, p, t):
    gathers_wait(p)

    @pl.when(j >= 1)
    def _():
      scatters_wait()

    @pl.when(j + 1 < _NB)
    def _():
      edge_wait()
      offset_src(1 - p)
      gathers_fire(1 - p)

    mul(p)

    @pl.when(j + 2 < _NB)
    def _():
      if t == 0:
        edge_fire(j + 2, p, 2)
      elif t == 1:
        edge_fire(j + 2, p, 0)
      else:
        edge_fire(j + 2, p, 1)

    scatters_fire(p, t)

  def body(j, _):
    for r in range(6):
      @pl.when(j % 6 == r)
      def _(r=r):
        step(j, r % 2, r % 3)
    return _

  lax.fori_loop(0, _NB, body, None)
  scatters_wait()
  plsc.subcore_barrier()

  @pl.when(s < 15)
  def _():
    pltpu.sync_copy(acc.at[pl.ds(s * _SLAB, _SLAB)],
                    y_ref.at[pl.ds(c * _N + s * _SLAB, _SLAB)])

  @pl.when(s == 15)
  def _():
    pltpu.sync_copy(acc.at[pl.ds(15 * _SLAB, _SLAB_LAST)],
                    y_ref.at[pl.ds(c * _N + 15 * _SLAB, _SLAB_LAST)])


@functools.lru_cache(maxsize=None)
def _make_spmm():
  return pl.kernel(
    _spmm_body,
    out_type=jax.ShapeDtypeStruct((_NC * _N, _H), jnp.float32),
    mesh=plsc.VectorSubcoreMesh(core_axis_name="c", subcore_axis_name="s",
                                num_cores=_NC, num_subcores=_NS),
    compiler_params=pltpu.CompilerParams(use_tc_tiling_on_sc=False),
    scratch_types=[
        pltpu.VMEM((_KB, _CH), jnp.int32),
        pltpu.VMEM((_KB, _CH), jnp.int32),
        pltpu.VMEM((_BB,), jnp.float32),
        pltpu.VMEM((_BB,), jnp.float32),
        pltpu.VMEM((_KB, _CH), jnp.int32),
        pltpu.VMEM((_KB, _CH), jnp.int32),
        pltpu.VMEM((_KB, _CH), jnp.int32),
        pltpu.VMEM((_BB, _H), jnp.float32),
        pltpu.VMEM((_BB, _H), jnp.float32),
        pltpu.VMEM_SHARED((_N, _H), jnp.float32),
        pltpu.SemaphoreType.DMA,
        pltpu.SemaphoreType.DMA,
        pltpu.SemaphoreType.DMA,
    ],
  )


def _prep_edges(edge_index, edge_weight):
  pad = _EPAD - _E
  src4 = jnp.pad(edge_index[1], (0, pad)).reshape(_NS, _NB, _KB, _CH)
  dst4 = jnp.pad(edge_index[0], (0, pad)).reshape(_NS, _NB, _KB, _CH)
  w3 = jnp.pad(edge_weight, (0, pad)).reshape(_NS, _NB, _BB)
  return src4, dst4, w3


_FCH = 184


def _fuse_body_factory(scale):
  def _fuse_body(yb_ref, yc_ref, g_ref, acc_ref, xn_ref, ao_ref,
                 byb0, byc0, bg0, bacc0, bxn0, bao0,
                 byb1, byc1, bg1, bacc1, bxn1, bao1, isem, osem):
    c = lax.axis_index("c")
    s = lax.axis_index("s")
    row0 = c * _N + s * _SLAB
    g0 = s * _SLAB

    ins = ((byb0, byc0, bg0, bacc0), (byb1, byc1, bg1, bacc1))
    outs = ((bxn0, bao0), (bxn1, bao1))

    def in_fire(j, p):
      byb, byc, bg, bacc = ins[p]
      pltpu.async_copy(yb_ref.at[pl.ds(row0 + j * _FCH, _FCH)], byb, isem)
      pltpu.async_copy(yc_ref.at[pl.ds(row0 + j * _FCH, _FCH)], byc, isem)
      pltpu.async_copy(g_ref.at[pl.ds(g0 + j * _FCH, _FCH)], bg, isem)
      pltpu.async_copy(acc_ref.at[pl.ds(row0 + j * _FCH, _FCH)], bacc, isem)

    def in_wait(p):
      byb, byc, bg, bacc = ins[p]
      pltpu.make_async_copy(yb_ref.at[pl.ds(row0, _FCH)], byb, isem).wait()
      pltpu.make_async_copy(yc_ref.at[pl.ds(row0, _FCH)], byc, isem).wait()
      pltpu.make_async_copy(g_ref.at[pl.ds(g0, _FCH)], bg, isem).wait()
      pltpu.make_async_copy(acc_ref.at[pl.ds(row0, _FCH)], bacc, isem).wait()

    def out_fire(j, p):
      bxn, bao = outs[p]
      pltpu.async_copy(bxn, xn_ref.at[pl.ds(row0 + j * _FCH, _FCH)], osem)
      pltpu.async_copy(bao, ao_ref.at[pl.ds(row0 + j * _FCH, _FCH)], osem)

    def out_wait(p):
      bxn, bao = outs[p]
      pltpu.make_async_copy(bxn, xn_ref.at[pl.ds(row0, _FCH)], osem).wait()
      pltpu.make_async_copy(bao, ao_ref.at[pl.ds(row0, _FCH)], osem).wait()

    def compute(p, nrows):
      byb, byc, bg, bacc = ins[p]
      bxn, bao = outs[p]

      def rowfn(i, _):
        g = bg[i, pl.ds(0, 16)]
        bl = byb[i, pl.ds(0, 16)]
        bh = byb[i, pl.ds(16, 16)]
        cl = byc[i, pl.ds(0, 16)]
        ch = byc[i, pl.ds(16, 16)]
        nl = g * (bl - cl) + cl
        nh = g * (bh - ch) + ch
        bxn[i, pl.ds(0, 16)] = nl
        bxn[i, pl.ds(16, 16)] = nh
        al = bacc[i, pl.ds(0, 16)] + nl
        ah = bacc[i, pl.ds(16, 16)] + nh
        if scale != 1.0:
          al = al * scale
          ah = ah * scale
        bao[i, pl.ds(0, 16)] = al
        bao[i, pl.ds(16, 16)] = ah
        return _

      lax.fori_loop(0, nrows, rowfn, None)

    nch = _SLAB // _FCH
    nch_last = _SLAB_LAST // _FCH

    def run(nfull):
      in_fire(0, 0)

      def chunk(j, _):
        @pl.when(j % 2 == 0)
        def _():
          stepc(j, 0)

        @pl.when(j % 2 == 1)
        def _():
          stepc(j, 1)
        return _

      def stepc(j, p):
        in_wait(p)

        @pl.when(j + 1 < nfull)
        def _():
          in_fire(j + 1, 1 - p)

        @pl.when(j >= 2)
        def _():
          out_wait(p)

        compute(p, _FCH)
        out_fire(j, p)

      lax.fori_loop(0, nfull, chunk, None)
      if nfull >= 2:
        out_wait(nfull % 2)
      out_wait((nfull + 1) % 2)

    @pl.when(s < 15)
    def _():
      run(nch)

    @pl.when(s == 15)
    def _():
      run(nch_last)
      t0 = row0 + nch_last * _FCH
      gt0 = g0 + nch_last * _FCH
      byb, byc, bg, bacc = ins[0]
      bxn, bao = outs[0]
      pltpu.sync_copy(yb_ref.at[pl.ds(t0, 136)], byb.at[pl.ds(0, 136)])
      pltpu.sync_copy(yc_ref.at[pl.ds(t0, 136)], byc.at[pl.ds(0, 136)])
      pltpu.sync_copy(g_ref.at[pl.ds(gt0, 136)], bg.at[pl.ds(0, 136)])
      pltpu.sync_copy(acc_ref.at[pl.ds(t0, 136)], bacc.at[pl.ds(0, 136)])
      compute(0, 136)
      pltpu.sync_copy(bxn.at[pl.ds(0, 136)], xn_ref.at[pl.ds(t0, 136)])
      pltpu.sync_copy(bao.at[pl.ds(0, 136)], ao_ref.at[pl.ds(t0, 136)])

  return _fuse_body


@functools.lru_cache(maxsize=None)
def _make_fuse(scale):
  buf = lambda w: pltpu.VMEM((_FCH, w), jnp.float32)
  return pl.kernel(
      _fuse_body_factory(scale),
      out_type=(jax.ShapeDtypeStruct((_NC * _N, _H), jnp.float32),
                jax.ShapeDtypeStruct((_NC * _N, _H), jnp.float32)),
      mesh=plsc.VectorSubcoreMesh(core_axis_name="c", subcore_axis_name="s",
                                  num_cores=_NC, num_subcores=_NS),
      compiler_params=pltpu.CompilerParams(use_tc_tiling_on_sc=False),
      scratch_types=[
          buf(32), buf(32), buf(_H), buf(32), buf(32), buf(32),
          buf(32), buf(32), buf(_H), buf(32), buf(32), buf(32),
          pltpu.SemaphoreType.DMA,
          pltpu.SemaphoreType.DMA,
      ],
  )


_BLK = 1000
_GRID = _N // _BLK
_UBLKS = _N_USERS // _BLK


def _gate(gi, w1, b1, w2, b2):
  h = lax.dot_general(gi, w1, (((1,), (1,)), ((), ())),
                      preferred_element_type=jnp.float32)
  h = jnp.maximum(h + b1, 0.0)
  g = lax.dot_general(h, w2, (((1,), (1,)), ((), ())),
                      preferred_element_type=jnp.float32)
  return jax.nn.sigmoid(g + b2)


def _rep_w2(w2, b2, w=_H):
  return jnp.tile(w2, (w, 1)), jnp.tile(b2.reshape(1, 1), (1, w))


def _init_body(x1_ref, x2_ref, w1_ref, b1_ref, w2_ref, b2_ref, o_ref):
  x1 = x1_ref[...]
  x2 = x2_ref[...]
  g = _gate(jnp.concatenate([x1, x2], axis=1),
            w1_ref[...], b1_ref[...], w2_ref[...], b2_ref[...])
  out_lo = g * x1[:, :_H] + (1.0 - g) * x2[:, :_H]
  out_hi = g * x1[:, _H:] + (1.0 - g) * x2[:, _H:]
  o_ref[0] = out_lo
  o_ref[1] = out_hi


def _init_fuse(a, b, w1, b1, w2, b2):
  return pl.pallas_call(
      _init_body,
      grid=(_GRID,),
      in_specs=[
          pl.BlockSpec((_BLK, _D), lambda i: (i, 0)),
          pl.BlockSpec((_BLK, _D), lambda i: (i, 0)),
          pl.BlockSpec((32, 2 * _D), lambda i: (0, 0)),
          pl.BlockSpec((1, 32), lambda i: (0, 0)),
          pl.BlockSpec((_H, 32), lambda i: (0, 0)),
          pl.BlockSpec((1, _H), lambda i: (0, 0)),
      ],
      out_specs=pl.BlockSpec((2, _BLK, _H), lambda i: (0, i, 0)),
      out_shape=jax.ShapeDtypeStruct((2, _N, _H), jnp.float32),
  )(a, b, w1, b1, w2, b2)


def _gate_body(yb_ref, herb_ref, w1_ref, b1_ref, w2_ref, b2_ref, g_ref):
  g_ref[...] = _gate(
      jnp.concatenate([yb_ref[0], yb_ref[1], herb_ref[...]], axis=1),
      w1_ref[...], b1_ref[...], w2_ref[...], b2_ref[...])


def _item_gates(yb3, herb, w1, b1, w2, b2):
  return pl.pallas_call(
      _gate_body,
      grid=(_N_ITEMS // _BLK,),
      in_specs=[
          pl.BlockSpec((2, _BLK, _H), lambda i: (0, _UBLKS + i, 0)),
          pl.BlockSpec((_BLK, _D), lambda i: (i, 0)),
          pl.BlockSpec((32, 2 * _D), lambda i: (0, 0)),
          pl.BlockSpec((1, 32), lambda i: (0, 0)),
          pl.BlockSpec((_H, 32), lambda i: (0, 0)),
          pl.BlockSpec((1, _H), lambda i: (0, 0)),
      ],
      out_specs=pl.BlockSpec((_BLK, _H), lambda i: (i, 0)),
      out_shape=jax.ShapeDtypeStruct((_N_ITEMS, _H), jnp.float32),
  )(yb3, herb, w1, b1, w2, b2)


def kernel(users_emb, items_emb, symptom_emb, herb_emb, base_edge_index,
           base_edge_weight, cooccur_edge_index, cooccur_edge_weight,
           gate_W1, gate_b1, gate_W2, gate_b2):
  n_layers = gate_W1.shape[0] - 1

  a = jnp.concatenate([users_emb, items_emb], axis=0)
  b = jnp.concatenate([symptom_emb, herb_emb], axis=0)
  w2r, b2r = _rep_w2(gate_W2[0], gate_b2[0])
  all_emb = _init_fuse(a, b, gate_W1[0], gate_b1[0:1], w2r, b2r)

  base_edges = _prep_edges(base_edge_index, base_edge_weight)
  cooc_edges = _prep_edges(cooccur_edge_index, cooccur_edge_weight)

  spmm = _make_spmm()
  zrows = jnp.zeros((_SLAB, _H), jnp.float32)
  ones_g = jnp.ones((_N_USERS, _H), jnp.float32)
  xcur = all_emb.reshape(_NC * _N, _H)
  acc = xcur
  for layer in range(1, n_layers + 1):
    ybase = spmm(xcur, zrows, *base_edges)
    ycooc = spmm(xcur, zrows, *cooc_edges)
    w2r, b2r = _rep_w2(gate_W2[layer], gate_b2[layer])
    g_items = _item_gates(ybase.reshape(2, _N, _H), herb_emb,
                          gate_W1[layer], gate_b1[layer:layer + 1], w2r, b2r)
    gfull = jnp.concatenate([ones_g, g_items], axis=0)
    scale = 1.0 / (n_layers + 1) if layer == n_layers else 1.0
    xcur, acc = _make_fuse(scale)(ybase, ycooc, gfull, acc)

  users = jnp.concatenate([acc[:_N_USERS], acc[_N: _N + _N_USERS]], axis=1)
  items = jnp.concatenate([acc[_N_USERS:_N], acc[_N + _N_USERS:]], axis=1)
  return (users, items)


# speedup vs baseline: 1.2054x; 1.2039x over previous
import functools

import jax
import jax.numpy as jnp
from jax import lax
from jax.experimental import pallas as pl
from jax.experimental.pallas import tpu as pltpu
from jax.experimental.pallas import tpu_sc as plsc

_N_USERS = 25000
_N_ITEMS = 25000
_N = _N_USERS + _N_ITEMS
_D = 64
_H = 32
_E = 800000

_NC = 2
_NS = 16
_CH = 128
_KB = 3
_BB = _CH * _KB
_NB = 131
_EPT = _BB * _NB
_EPAD = _EPT * _NS

_SLAB = 3128
_SLAB_LAST = _N - 15 * _SLAB


def _spmm_body(x_ref, z_ref, src_ref, dst_ref, w_ref, y_ref,
               sidx0, sidx1, wbuf0, wbuf1, didx0, didx1, didx2,
               rows0, rows1, acc, esem, gsem, ssem):
  c = lax.axis_index("c")
  s = lax.axis_index("s")

  @pl.when(s < 15)
  def _():
    pltpu.sync_copy(z_ref, acc.at[pl.ds(s * _SLAB, _SLAB)])

  @pl.when(s == 15)
  def _():
    pltpu.sync_copy(z_ref.at[pl.ds(0, _SLAB_LAST)],
                    acc.at[pl.ds(15 * _SLAB, _SLAB_LAST)])

  plsc.subcore_barrier()

  sidx = (sidx0, sidx1)
  wbufs = (wbuf0, wbuf1)
  didx = (didx0, didx1, didx2)

  def edge_fire(j, p, t):
    pltpu.async_copy(src_ref.at[s, j], sidx[p], esem)
    pltpu.async_copy(dst_ref.at[s, j], didx[t], esem)
    pltpu.async_copy(w_ref.at[s, j], wbufs[p], esem)

  def edge_wait():
    pltpu.make_async_copy(src_ref.at[s, 0], sidx[0], esem).wait()
    pltpu.make_async_copy(dst_ref.at[s, 0], didx[0], esem).wait()
    pltpu.make_async_copy(w_ref.at[s, 0], wbufs[0], esem).wait()

  coff = c * _N

  def offset_src(p):
    for k in range(_KB):
      for q in range(_CH // 16):
        v = sidx[p][k, pl.ds(q * 16, 16)]
        sidx[p][k, pl.ds(q * 16, 16)] = v + coff

  def gathers_fire(p):
    for k in range(_KB):
      pltpu.async_copy(x_ref.at[sidx[p].at[k]],
                       rows0.at[pl.ds(k * _CH, _CH)] if p == 0
                       else rows1.at[pl.ds(k * _CH, _CH)], gsem)

  def gathers_wait(p):
    for k in range(_KB):
      pltpu.make_async_copy(x_ref.at[sidx[p].at[k]],
                            rows0.at[pl.ds(k * _CH, _CH)] if p == 0
                            else rows1.at[pl.ds(k * _CH, _CH)], gsem).wait()

  def scatters_fire(p, t):
    rows = rows0 if p == 0 else rows1
    for k in range(_KB):
      pltpu.async_copy(rows.at[pl.ds(k * _CH, _CH)],
                       acc.at[didx[t].at[k]], ssem, add=True)

  def scatters_wait():
    for k in range(_KB):
      pltpu.make_async_copy(rows0.at[pl.ds(k * _CH, _CH)],
                            acc.at[didx[0].at[k]], ssem).wait()

  def mul(p):
    rows = rows0 if p == 0 else rows1
    wb = wbufs[p]

    def _mul(m, _):
      w16 = wb[pl.ds(m * 16, 16)]
      for u in range(16):
        e = m * 16 + u
        w = w16[u]
        rows[e, pl.ds(0, 16)] = rows[e, pl.ds(0, 16)] * w
        rows[e, pl.ds(16, 16)] = rows[e, pl.ds(16, 16)] * w
      return _

    lax.fori_loop(0, _BB // 16, _mul, None)

  edge_fire(0, 0, 0)
  edge_wait()
  offset_src(0)
  gathers_fire(0)
  edge_fire(1, 1, 1)

  def step(j, p, t):
    gathers_wait(p)

    @pl.when(j >= 1)
    def _():
      scatters_wait()

    @pl.when(j + 1 < _NB)
    def _():
      edge_wait()
      offset_src(1 - p)
      gathers_fire(1 - p)

    mul(p)

    @pl.when(j + 2 < _NB)
    def _():
      if t == 0:
        edge_fire(j + 2, p, 2)
      elif t == 1:
        edge_fire(j + 2, p, 0)
      else:
        edge_fire(j + 2, p, 1)

    scatters_fire(p, t)

  def body(j, _):
    for r in range(6):
      @pl.when(j % 6 == r)
      def _(r=r):
        step(j, r % 2, r % 3)
    return _

  lax.fori_loop(0, _NB, body, None)
  scatters_wait()
  plsc.subcore_barrier()

  @pl.when(s < 15)
  def _():
    pltpu.sync_copy(acc.at[pl.ds(s * _SLAB, _SLAB)],
                    y_ref.at[pl.ds(c * _N + s * _SLAB, _SLAB)])

  @pl.when(s == 15)
  def _():
    pltpu.sync_copy(acc.at[pl.ds(15 * _SLAB, _SLAB_LAST)],
                    y_ref.at[pl.ds(c * _N + 15 * _SLAB, _SLAB_LAST)])


@functools.lru_cache(maxsize=None)
def _make_spmm():
  return pl.kernel(
    _spmm_body,
    out_type=jax.ShapeDtypeStruct((_NC * _N, _H), jnp.float32),
    mesh=plsc.VectorSubcoreMesh(core_axis_name="c", subcore_axis_name="s",
                                num_cores=_NC, num_subcores=_NS),
    compiler_params=pltpu.CompilerParams(use_tc_tiling_on_sc=False),
    scratch_types=[
        pltpu.VMEM((_KB, _CH), jnp.int32),
        pltpu.VMEM((_KB, _CH), jnp.int32),
        pltpu.VMEM((_BB,), jnp.float32),
        pltpu.VMEM((_BB,), jnp.float32),
        pltpu.VMEM((_KB, _CH), jnp.int32),
        pltpu.VMEM((_KB, _CH), jnp.int32),
        pltpu.VMEM((_KB, _CH), jnp.int32),
        pltpu.VMEM((_BB, _H), jnp.float32),
        pltpu.VMEM((_BB, _H), jnp.float32),
        pltpu.VMEM_SHARED((_N, _H), jnp.float32),
        pltpu.SemaphoreType.DMA,
        pltpu.SemaphoreType.DMA,
        pltpu.SemaphoreType.DMA,
    ],
  )


def _prep_edges(edge_index, edge_weight):
  pad = _EPAD - _E
  fill = jnp.arange(pad, dtype=jnp.int32)
  src4 = jnp.concatenate([edge_index[1], fill]).reshape(_NS, _NB, _KB, _CH)
  dst4 = jnp.concatenate([edge_index[0], fill]).reshape(_NS, _NB, _KB, _CH)
  w3 = jnp.pad(edge_weight, (0, pad)).reshape(_NS, _NB, _BB)
  return src4, dst4, w3


_FCH = 184


def _fuse_body_factory(scale):
  def _fuse_body(yb_ref, yc_ref, g_ref, acc_ref, xn_ref, ao_ref,
                 byb0, byc0, bg0, bacc0, bxn0, bao0,
                 byb1, byc1, bg1, bacc1, bxn1, bao1, isem, osem):
    c = lax.axis_index("c")
    s = lax.axis_index("s")
    row0 = c * _N + s * _SLAB
    g0 = s * _SLAB

    ins = ((byb0, byc0, bg0, bacc0), (byb1, byc1, bg1, bacc1))
    outs = ((bxn0, bao0), (bxn1, bao1))

    def in_fire(j, p):
      byb, byc, bg, bacc = ins[p]
      pltpu.async_copy(yb_ref.at[pl.ds(row0 + j * _FCH, _FCH)], byb, isem)
      pltpu.async_copy(yc_ref.at[pl.ds(row0 + j * _FCH, _FCH)], byc, isem)
      pltpu.async_copy(g_ref.at[pl.ds(g0 + j * _FCH, _FCH)], bg, isem)
      pltpu.async_copy(acc_ref.at[pl.ds(row0 + j * _FCH, _FCH)], bacc, isem)

    def in_wait(p):
      byb, byc, bg, bacc = ins[p]
      pltpu.make_async_copy(yb_ref.at[pl.ds(row0, _FCH)], byb, isem).wait()
      pltpu.make_async_copy(yc_ref.at[pl.ds(row0, _FCH)], byc, isem).wait()
      pltpu.make_async_copy(g_ref.at[pl.ds(g0, _FCH)], bg, isem).wait()
      pltpu.make_async_copy(acc_ref.at[pl.ds(row0, _FCH)], bacc, isem).wait()

    def out_fire(j, p):
      bxn, bao = outs[p]
      pltpu.async_copy(bxn, xn_ref.at[pl.ds(row0 + j * _FCH, _FCH)], osem)
      pltpu.async_copy(bao, ao_ref.at[pl.ds(row0 + j * _FCH, _FCH)], osem)

    def out_wait(p):
      bxn, bao = outs[p]
      pltpu.make_async_copy(bxn, xn_ref.at[pl.ds(row0, _FCH)], osem).wait()
      pltpu.make_async_copy(bao, ao_ref.at[pl.ds(row0, _FCH)], osem).wait()

    def compute(p, nrows):
      byb, byc, bg, bacc = ins[p]
      bxn, bao = outs[p]

      def rowfn(i, _):
        g = bg[i, pl.ds(0, 16)]
        bl = byb[i, pl.ds(0, 16)]
        bh = byb[i, pl.ds(16, 16)]
        cl = byc[i, pl.ds(0, 16)]
        ch = byc[i, pl.ds(16, 16)]
        nl = g * (bl - cl) + cl
        nh = g * (bh - ch) + ch
        bxn[i, pl.ds(0, 16)] = nl
        bxn[i, pl.ds(16, 16)] = nh
        al = bacc[i, pl.ds(0, 16)] + nl
        ah = bacc[i, pl.ds(16, 16)] + nh
        if scale != 1.0:
          al = al * scale
          ah = ah * scale
        bao[i, pl.ds(0, 16)] = al
        bao[i, pl.ds(16, 16)] = ah
        return _

      lax.fori_loop(0, nrows, rowfn, None)

    nch = _SLAB // _FCH
    nch_last = _SLAB_LAST // _FCH

    def run(nfull):
      in_fire(0, 0)

      def chunk(j, _):
        @pl.when(j % 2 == 0)
        def _():
          stepc(j, 0)

        @pl.when(j % 2 == 1)
        def _():
          stepc(j, 1)
        return _

      def stepc(j, p):
        in_wait(p)

        @pl.when(j + 1 < nfull)
        def _():
          in_fire(j + 1, 1 - p)

        @pl.when(j >= 2)
        def _():
          out_wait(p)

        compute(p, _FCH)
        out_fire(j, p)

      lax.fori_loop(0, nfull, chunk, None)
      if nfull >= 2:
        out_wait(nfull % 2)
      out_wait((nfull + 1) % 2)

    @pl.when(s < 15)
    def _():
      run(nch)

    @pl.when(s == 15)
    def _():
      run(nch_last)
      t0 = row0 + nch_last * _FCH
      gt0 = g0 + nch_last * _FCH
      byb, byc, bg, bacc = ins[0]
      bxn, bao = outs[0]
      pltpu.sync_copy(yb_ref.at[pl.ds(t0, 136)], byb.at[pl.ds(0, 136)])
      pltpu.sync_copy(yc_ref.at[pl.ds(t0, 136)], byc.at[pl.ds(0, 136)])
      pltpu.sync_copy(g_ref.at[pl.ds(gt0, 136)], bg.at[pl.ds(0, 136)])
      pltpu.sync_copy(acc_ref.at[pl.ds(t0, 136)], bacc.at[pl.ds(0, 136)])
      compute(0, 136)
      pltpu.sync_copy(bxn.at[pl.ds(0, 136)], xn_ref.at[pl.ds(t0, 136)])
      pltpu.sync_copy(bao.at[pl.ds(0, 136)], ao_ref.at[pl.ds(t0, 136)])

  return _fuse_body


@functools.lru_cache(maxsize=None)
def _make_fuse(scale):
  buf = lambda w: pltpu.VMEM((_FCH, w), jnp.float32)
  return pl.kernel(
      _fuse_body_factory(scale),
      out_type=(jax.ShapeDtypeStruct((_NC * _N, _H), jnp.float32),
                jax.ShapeDtypeStruct((_NC * _N, _H), jnp.float32)),
      mesh=plsc.VectorSubcoreMesh(core_axis_name="c", subcore_axis_name="s",
                                  num_cores=_NC, num_subcores=_NS),
      compiler_params=pltpu.CompilerParams(use_tc_tiling_on_sc=False),
      scratch_types=[
          buf(32), buf(32), buf(_H), buf(32), buf(32), buf(32),
          buf(32), buf(32), buf(_H), buf(32), buf(32), buf(32),
          pltpu.SemaphoreType.DMA,
          pltpu.SemaphoreType.DMA,
      ],
  )


_BLK = 1000
_GRID = _N // _BLK
_UBLKS = _N_USERS // _BLK


def _gate(gi, w1, b1, w2, b2):
  h = lax.dot_general(gi, w1, (((1,), (1,)), ((), ())),
                      preferred_element_type=jnp.float32)
  h = jnp.maximum(h + b1, 0.0)
  g = lax.dot_general(h, w2, (((1,), (1,)), ((), ())),
                      preferred_element_type=jnp.float32)
  return jax.nn.sigmoid(g + b2)


def _rep_w2(w2, b2, w=_H):
  return jnp.tile(w2, (w, 1)), jnp.tile(b2.reshape(1, 1), (1, w))


def _init_body(x1_ref, x2_ref, w1_ref, b1_ref, w2_ref, b2_ref, o_ref):
  x1 = x1_ref[...]
  x2 = x2_ref[...]
  g = _gate(jnp.concatenate([x1, x2], axis=1),
            w1_ref[...], b1_ref[...], w2_ref[...], b2_ref[...])
  out_lo = g * x1[:, :_H] + (1.0 - g) * x2[:, :_H]
  out_hi = g * x1[:, _H:] + (1.0 - g) * x2[:, _H:]
  o_ref[0] = out_lo
  o_ref[1] = out_hi


def _init_fuse(a, b, w1, b1, w2, b2):
  return pl.pallas_call(
      _init_body,
      grid=(_GRID,),
      in_specs=[
          pl.BlockSpec((_BLK, _D), lambda i: (i, 0)),
          pl.BlockSpec((_BLK, _D), lambda i: (i, 0)),
          pl.BlockSpec((32, 2 * _D), lambda i: (0, 0)),
          pl.BlockSpec((1, 32), lambda i: (0, 0)),
          pl.BlockSpec((_H, 32), lambda i: (0, 0)),
          pl.BlockSpec((1, _H), lambda i: (0, 0)),
      ],
      out_specs=pl.BlockSpec((2, _BLK, _H), lambda i: (0, i, 0)),
      out_shape=jax.ShapeDtypeStruct((2, _N, _H), jnp.float32),
  )(a, b, w1, b1, w2, b2)


def _gate_body(yb_ref, herb_ref, w1_ref, b1_ref, w2_ref, b2_ref, g_ref):
  g_ref[...] = _gate(
      jnp.concatenate([yb_ref[0], yb_ref[1], herb_ref[...]], axis=1),
      w1_ref[...], b1_ref[...], w2_ref[...], b2_ref[...])


def _item_gates(yb3, herb, w1, b1, w2, b2):
  return pl.pallas_call(
      _gate_body,
      grid=(_N_ITEMS // _BLK,),
      in_specs=[
          pl.BlockSpec((2, _BLK, _H), lambda i: (0, _UBLKS + i, 0)),
          pl.BlockSpec((_BLK, _D), lambda i: (i, 0)),
          pl.BlockSpec((32, 2 * _D), lambda i: (0, 0)),
          pl.BlockSpec((1, 32), lambda i: (0, 0)),
          pl.BlockSpec((_H, 32), lambda i: (0, 0)),
          pl.BlockSpec((1, _H), lambda i: (0, 0)),
      ],
      out_specs=pl.BlockSpec((_BLK, _H), lambda i: (i, 0)),
      out_shape=jax.ShapeDtypeStruct((_N_ITEMS, _H), jnp.float32),
  )(yb3, herb, w1, b1, w2, b2)


def kernel(users_emb, items_emb, symptom_emb, herb_emb, base_edge_index,
           base_edge_weight, cooccur_edge_index, cooccur_edge_weight,
           gate_W1, gate_b1, gate_W2, gate_b2):
  n_layers = gate_W1.shape[0] - 1

  a = jnp.concatenate([users_emb, items_emb], axis=0)
  b = jnp.concatenate([symptom_emb, herb_emb], axis=0)
  w2r, b2r = _rep_w2(gate_W2[0], gate_b2[0])
  all_emb = _init_fuse(a, b, gate_W1[0], gate_b1[0:1], w2r, b2r)

  base_edges = _prep_edges(base_edge_index, base_edge_weight)
  cooc_edges = _prep_edges(cooccur_edge_index, cooccur_edge_weight)

  spmm = _make_spmm()
  zrows = jnp.zeros((_SLAB, _H), jnp.float32)
  ones_g = jnp.ones((_N_USERS, _H), jnp.float32)
  xcur = all_emb.reshape(_NC * _N, _H)
  acc = xcur
  for layer in range(1, n_layers + 1):
    ybase = spmm(xcur, zrows, *base_edges)
    ycooc = spmm(xcur, zrows, *cooc_edges)
    w2r, b2r = _rep_w2(gate_W2[layer], gate_b2[layer])
    g_items = _item_gates(ybase.reshape(2, _N, _H), herb_emb,
                          gate_W1[layer], gate_b1[layer:layer + 1], w2r, b2r)
    gfull = jnp.concatenate([ones_g, g_items], axis=0)
    scale = 1.0 / (n_layers + 1) if layer == n_layers else 1.0
    xcur, acc = _make_fuse(scale)(ybase, ycooc, gfull, acc)

  users = jnp.concatenate([acc[:_N_USERS], acc[_N: _N + _N_USERS]], axis=1)
  items = jnp.concatenate([acc[_N_USERS:_N], acc[_N + _N_USERS:]], axis=1)
  return (users, items)


# speedup vs baseline: 1.2654x; 1.0498x over previous
import functools

import jax
import jax.numpy as jnp
from jax import lax
from jax.experimental import pallas as pl
from jax.experimental.pallas import tpu as pltpu
from jax.experimental.pallas import tpu_sc as plsc

_N_USERS = 25000
_N_ITEMS = 25000
_N = _N_USERS + _N_ITEMS
_D = 64
_H = 32
_E = 800000

_NC = 2
_NS = 16
_CH = 128
_KB = 3
_BB = _CH * _KB
_NB = 131
_EPT = _BB * _NB
_EPAD = _EPT * _NS

_SLAB = 3128
_SLAB_LAST = _N - 15 * _SLAB


def _spmm_body(x_ref, z_ref, src_ref, dst_ref, w_ref, y_ref,
               sidx0, sidx1, wbuf0, wbuf1, didx0, didx1, didx2,
               rows0, rows1, rowsf, acc, esem, gsem, ssem):
  c = lax.axis_index("c")
  s = lax.axis_index("s")

  @pl.when(s < 15)
  def _():
    pltpu.sync_copy(z_ref, acc.at[pl.ds(s * _SLAB, _SLAB)])

  @pl.when(s == 15)
  def _():
    pltpu.sync_copy(z_ref.at[pl.ds(0, _SLAB_LAST)],
                    acc.at[pl.ds(15 * _SLAB, _SLAB_LAST)])

  plsc.subcore_barrier()

  sidx = (sidx0, sidx1)
  wbufs = (wbuf0, wbuf1)
  didx = (didx0, didx1, didx2)

  def edge_fire(j, p, t):
    pltpu.async_copy(src_ref.at[s, j], sidx[p], esem)
    pltpu.async_copy(dst_ref.at[s, j], didx[t], esem)
    pltpu.async_copy(w_ref.at[s, j], wbufs[p], esem)

  def edge_wait():
    pltpu.make_async_copy(src_ref.at[s, 0], sidx[0], esem).wait()
    pltpu.make_async_copy(dst_ref.at[s, 0], didx[0], esem).wait()
    pltpu.make_async_copy(w_ref.at[s, 0], wbufs[0], esem).wait()

  coff = c * _N

  def offset_src(p):
    for k in range(_KB):
      for q in range(_CH // 16):
        v = sidx[p][k, pl.ds(q * 16, 16)]
        sidx[p][k, pl.ds(q * 16, 16)] = v + coff

  def gathers_fire(p):
    for k in range(_KB):
      pltpu.async_copy(x_ref.at[sidx[p].at[k]],
                       rows0.at[pl.ds(k * _CH, _CH)] if p == 0
                       else rows1.at[pl.ds(k * _CH, _CH)], gsem)

  def gathers_wait(p):
    for k in range(_KB):
      pltpu.make_async_copy(x_ref.at[sidx[p].at[k]],
                            rows0.at[pl.ds(k * _CH, _CH)] if p == 0
                            else rows1.at[pl.ds(k * _CH, _CH)], gsem).wait()

  def scatters_fire(t):
    for k in range(_KB):
      pltpu.async_copy(rowsf.at[pl.ds(k * _CH, _CH)],
                       acc.at[didx[t].at[k]], ssem, add=True)

  def scatters_wait():
    for k in range(_KB):
      pltpu.make_async_copy(rowsf.at[pl.ds(k * _CH, _CH)],
                            acc.at[didx[0].at[k]], ssem).wait()

  himask = jnp.full((16,), 0xFFFF0000, jnp.uint32)

  def mul(p):
    rows = rows0 if p == 0 else rows1
    wb = wbufs[p]

    def _mul(m, _):
      w16 = wb[pl.ds(m * 16, 16)]
      for u in range(16):
        e = m * 16 + u
        w = w16[u]
        r = rows[e, pl.ds(0, 16)]
        lo = lax.bitcast_convert_type(r << 16, jnp.float32)
        hi = lax.bitcast_convert_type(r & himask, jnp.float32)
        rowsf[e, pl.ds(0, 16)] = lo * w
        rowsf[e, pl.ds(16, 16)] = hi * w
      return _

    lax.fori_loop(0, _BB // 16, _mul, None)

  edge_fire(0, 0, 0)
  edge_wait()
  offset_src(0)
  gathers_fire(0)
  edge_fire(1, 1, 1)

  def step(j, p, t):
    gathers_wait(p)

    @pl.when(j >= 1)
    def _():
      scatters_wait()

    @pl.when(j + 1 < _NB)
    def _():
      edge_wait()
      offset_src(1 - p)
      gathers_fire(1 - p)


    @pl.when(j + 2 < _NB)
    def _():
      if t == 0:
        edge_fire(j + 2, p, 2)
      elif t == 1:
        edge_fire(j + 2, p, 0)
      else:
        edge_fire(j + 2, p, 1)

    scatters_fire(t)

  def body(j, _):
    for r in range(6):
      @pl.when(j % 6 == r)
      def _(r=r):
        step(j, r % 2, r % 3)
    return _

  lax.fori_loop(0, _NB, body, None)
  scatters_wait()
  plsc.subcore_barrier()

  @pl.when(s < 15)
  def _():
    pltpu.sync_copy(acc.at[pl.ds(s * _SLAB, _SLAB)],
                    y_ref.at[pl.ds(c * _N + s * _SLAB, _SLAB)])

  @pl.when(s == 15)
  def _():
    pltpu.sync_copy(acc.at[pl.ds(15 * _SLAB, _SLAB_LAST)],
                    y_ref.at[pl.ds(c * _N + 15 * _SLAB, _SLAB_LAST)])


@functools.lru_cache(maxsize=None)
def _make_spmm():
  return pl.kernel(
    _spmm_body,
    out_type=jax.ShapeDtypeStruct((_NC * _N, _H), jnp.float32),
    mesh=plsc.VectorSubcoreMesh(core_axis_name="c", subcore_axis_name="s",
                                num_cores=_NC, num_subcores=_NS),
    compiler_params=pltpu.CompilerParams(use_tc_tiling_on_sc=False),
    scratch_types=[
        pltpu.VMEM((_KB, _CH), jnp.int32),
        pltpu.VMEM((_KB, _CH), jnp.int32),
        pltpu.VMEM((_BB,), jnp.float32),
        pltpu.VMEM((_BB,), jnp.float32),
        pltpu.VMEM((_KB, _CH), jnp.int32),
        pltpu.VMEM((_KB, _CH), jnp.int32),
        pltpu.VMEM((_KB, _CH), jnp.int32),
        pltpu.VMEM((_BB, 16), jnp.uint32),
        pltpu.VMEM((_BB, 16), jnp.uint32),
        pltpu.VMEM((_BB, _H), jnp.float32),
        pltpu.VMEM_SHARED((_N, _H), jnp.float32),
        pltpu.SemaphoreType.DMA,
        pltpu.SemaphoreType.DMA,
        pltpu.SemaphoreType.DMA,
    ],
  )


def _prep_edges(edge_index, edge_weight):
  pad = _EPAD - _E
  fill = jnp.arange(pad, dtype=jnp.int32)
  src4 = jnp.concatenate([edge_index[1], fill]).reshape(_NS, _NB, _KB, _CH)
  dst4 = jnp.concatenate([edge_index[0], fill]).reshape(_NS, _NB, _KB, _CH)
  w3 = jnp.pad(edge_weight, (0, pad)).reshape(_NS, _NB, _BB)
  return src4, dst4, w3


_FCH = 184


def _fuse_body_factory(scale):
  def _fuse_body(yb_ref, yc_ref, g_ref, acc_ref, xn_ref, ao_ref,
                 byb0, byc0, bg0, bacc0, bxn0, bao0,
                 byb1, byc1, bg1, bacc1, bxn1, bao1, isem, osem):
    c = lax.axis_index("c")
    s = lax.axis_index("s")
    row0 = c * _N + s * _SLAB
    g0 = s * _SLAB

    ins = ((byb0, byc0, bg0, bacc0), (byb1, byc1, bg1, bacc1))
    outs = ((bxn0, bao0), (bxn1, bao1))

    def in_fire(j, p):
      byb, byc, bg, bacc = ins[p]
      pltpu.async_copy(yb_ref.at[pl.ds(row0 + j * _FCH, _FCH)], byb, isem)
      pltpu.async_copy(yc_ref.at[pl.ds(row0 + j * _FCH, _FCH)], byc, isem)
      pltpu.async_copy(g_ref.at[pl.ds(g0 + j * _FCH, _FCH)], bg, isem)
      pltpu.async_copy(acc_ref.at[pl.ds(row0 + j * _FCH, _FCH)], bacc, isem)

    def in_wait(p):
      byb, byc, bg, bacc = ins[p]
      pltpu.make_async_copy(yb_ref.at[pl.ds(row0, _FCH)], byb, isem).wait()
      pltpu.make_async_copy(yc_ref.at[pl.ds(row0, _FCH)], byc, isem).wait()
      pltpu.make_async_copy(g_ref.at[pl.ds(g0, _FCH)], bg, isem).wait()
      pltpu.make_async_copy(acc_ref.at[pl.ds(row0, _FCH)], bacc, isem).wait()

    def out_fire(j, p):
      bxn, bao = outs[p]
      pltpu.async_copy(bxn, xn_ref.at[pl.ds(row0 + j * _FCH, _FCH)], osem)
      pltpu.async_copy(bao, ao_ref.at[pl.ds(row0 + j * _FCH, _FCH)], osem)

    def out_wait(p):
      bxn, bao = outs[p]
      pltpu.make_async_copy(bxn, xn_ref.at[pl.ds(row0, _FCH)], osem).wait()
      pltpu.make_async_copy(bao, ao_ref.at[pl.ds(row0, _FCH)], osem).wait()

    def compute(p, nrows):
      byb, byc, bg, bacc = ins[p]
      bxn, bao = outs[p]

      rnd = jnp.full((16,), 0x8000, jnp.uint32)
      him = jnp.full((16,), 0xFFFF0000, jnp.uint32)

      def rowfn(i, _):
        g = bg[i, pl.ds(0, 16)]
        bl = byb[i, pl.ds(0, 16)]
        bh = byb[i, pl.ds(16, 16)]
        cl = byc[i, pl.ds(0, 16)]
        ch = byc[i, pl.ds(16, 16)]
        nl = g * (bl - cl) + cl
        nh = g * (bh - ch) + ch
        ul = lax.bitcast_convert_type(nl, jnp.uint32) + rnd
        uh = lax.bitcast_convert_type(nh, jnp.uint32) + rnd
        bxn[i, pl.ds(0, 16)] = (ul >> 16) | (uh & him)
        al = bacc[i, pl.ds(0, 16)] + nl
        ah = bacc[i, pl.ds(16, 16)] + nh
        if scale != 1.0:
          al = al * scale
          ah = ah * scale
        bao[i, pl.ds(0, 16)] = al
        bao[i, pl.ds(16, 16)] = ah
        return _

      lax.fori_loop(0, nrows, rowfn, None)

    nch = _SLAB // _FCH
    nch_last = _SLAB_LAST // _FCH

    def run(nfull):
      in_fire(0, 0)

      def chunk(j, _):
        @pl.when(j % 2 == 0)
        def _():
          stepc(j, 0)

        @pl.when(j % 2 == 1)
        def _():
          stepc(j, 1)
        return _

      def stepc(j, p):
        in_wait(p)

        @pl.when(j + 1 < nfull)
        def _():
          in_fire(j + 1, 1 - p)

        @pl.when(j >= 2)
        def _():
          out_wait(p)

        compute(p, _FCH)
        out_fire(j, p)

      lax.fori_loop(0, nfull, chunk, None)
      if nfull >= 2:
        out_wait(nfull % 2)
      out_wait((nfull + 1) % 2)

    @pl.when(s < 15)
    def _():
      run(nch)

    @pl.when(s == 15)
    def _():
      run(nch_last)
      t0 = row0 + nch_last * _FCH
      gt0 = g0 + nch_last * _FCH
      byb, byc, bg, bacc = ins[0]
      bxn, bao = outs[0]
      pltpu.sync_copy(yb_ref.at[pl.ds(t0, 136)], byb.at[pl.ds(0, 136)])
      pltpu.sync_copy(yc_ref.at[pl.ds(t0, 136)], byc.at[pl.ds(0, 136)])
      pltpu.sync_copy(g_ref.at[pl.ds(gt0, 136)], bg.at[pl.ds(0, 136)])
      pltpu.sync_copy(acc_ref.at[pl.ds(t0, 136)], bacc.at[pl.ds(0, 136)])
      compute(0, 136)
      pltpu.sync_copy(bxn.at[pl.ds(0, 136)], xn_ref.at[pl.ds(t0, 136)])
      pltpu.sync_copy(bao.at[pl.ds(0, 136)], ao_ref.at[pl.ds(t0, 136)])

  return _fuse_body


@functools.lru_cache(maxsize=None)
def _make_fuse(scale):
  buf = lambda w: pltpu.VMEM((_FCH, w), jnp.float32)
  pkbuf = pltpu.VMEM((_FCH, 16), jnp.uint32)
  return pl.kernel(
      _fuse_body_factory(scale),
      out_type=(jax.ShapeDtypeStruct((_NC * _N, 16), jnp.uint32),
                jax.ShapeDtypeStruct((_NC * _N, _H), jnp.float32)),
      mesh=plsc.VectorSubcoreMesh(core_axis_name="c", subcore_axis_name="s",
                                  num_cores=_NC, num_subcores=_NS),
      compiler_params=pltpu.CompilerParams(use_tc_tiling_on_sc=False),
      scratch_types=[
          buf(32), buf(32), buf(_H), buf(32), pkbuf, buf(32),
          buf(32), buf(32), buf(_H), buf(32), pkbuf, buf(32),
          pltpu.SemaphoreType.DMA,
          pltpu.SemaphoreType.DMA,
      ],
  )


_BLK = 1000
_GRID = _N // _BLK
_UBLKS = _N_USERS // _BLK


def _gate(gi, w1, b1, w2, b2):
  h = lax.dot_general(gi, w1, (((1,), (1,)), ((), ())),
                      preferred_element_type=jnp.float32)
  h = jnp.maximum(h + b1, 0.0)
  g = lax.dot_general(h, w2, (((1,), (1,)), ((), ())),
                      preferred_element_type=jnp.float32)
  return jax.nn.sigmoid(g + b2)


def _rep_w2(w2, b2, w=_H):
  return jnp.tile(w2, (w, 1)), jnp.tile(b2.reshape(1, 1), (1, w))


def _pack_bf16(nl, nh):
  ul = lax.bitcast_convert_type(nl, jnp.uint32) + jnp.uint32(0x8000)
  uh = lax.bitcast_convert_type(nh, jnp.uint32) + jnp.uint32(0x8000)
  return (ul >> 16) | (uh & jnp.uint32(0xFFFF0000))


def _init_body(x1_ref, x2_ref, w1_ref, b1_ref, w2_ref, b2_ref, o_ref, pk_ref):
  x1 = x1_ref[...]
  x2 = x2_ref[...]
  g = _gate(jnp.concatenate([x1, x2], axis=1),
            w1_ref[...], b1_ref[...], w2_ref[...], b2_ref[...])
  out_lo = g * x1[:, :_H] + (1.0 - g) * x2[:, :_H]
  out_hi = g * x1[:, _H:] + (1.0 - g) * x2[:, _H:]
  o_ref[0] = out_lo
  o_ref[1] = out_hi
  pk_ref[0] = _pack_bf16(out_lo[:, :16], out_lo[:, 16:])
  pk_ref[1] = _pack_bf16(out_hi[:, :16], out_hi[:, 16:])


def _init_fuse(a, b, w1, b1, w2, b2):
  return pl.pallas_call(
      _init_body,
      grid=(_GRID,),
      in_specs=[
          pl.BlockSpec((_BLK, _D), lambda i: (i, 0)),
          pl.BlockSpec((_BLK, _D), lambda i: (i, 0)),
          pl.BlockSpec((32, 2 * _D), lambda i: (0, 0)),
          pl.BlockSpec((1, 32), lambda i: (0, 0)),
          pl.BlockSpec((_H, 32), lambda i: (0, 0)),
          pl.BlockSpec((1, _H), lambda i: (0, 0)),
      ],
      out_specs=[
          pl.BlockSpec((2, _BLK, _H), lambda i: (0, i, 0)),
          pl.BlockSpec((2, _BLK, 16), lambda i: (0, i, 0)),
      ],
      out_shape=[
          jax.ShapeDtypeStruct((2, _N, _H), jnp.float32),
          jax.ShapeDtypeStruct((2, _N, 16), jnp.uint32),
      ],
  )(a, b, w1, b1, w2, b2)


def _gate_body(yb_ref, herb_ref, w1_ref, b1_ref, w2_ref, b2_ref, g_ref):
  g_ref[...] = _gate(
      jnp.concatenate([yb_ref[0], yb_ref[1], herb_ref[...]], axis=1),
      w1_ref[...], b1_ref[...], w2_ref[...], b2_ref[...])


def _item_gates(yb3, herb, w1, b1, w2, b2):
  return pl.pallas_call(
      _gate_body,
      grid=(_N_ITEMS // _BLK,),
      in_specs=[
          pl.BlockSpec((2, _BLK, _H), lambda i: (0, _UBLKS + i, 0)),
          pl.BlockSpec((_BLK, _D), lambda i: (i, 0)),
          pl.BlockSpec((32, 2 * _D), lambda i: (0, 0)),
          pl.BlockSpec((1, 32), lambda i: (0, 0)),
          pl.BlockSpec((_H, 32), lambda i: (0, 0)),
          pl.BlockSpec((1, _H), lambda i: (0, 0)),
      ],
      out_specs=pl.BlockSpec((_BLK, _H), lambda i: (i, 0)),
      out_shape=jax.ShapeDtypeStruct((_N_ITEMS, _H), jnp.float32),
  )(yb3, herb, w1, b1, w2, b2)


def kernel(users_emb, items_emb, symptom_emb, herb_emb, base_edge_index,
           base_edge_weight, cooccur_edge_index, cooccur_edge_weight,
           gate_W1, gate_b1, gate_W2, gate_b2):
  n_layers = gate_W1.shape[0] - 1

  a = jnp.concatenate([users_emb, items_emb], axis=0)
  b = jnp.concatenate([symptom_emb, herb_emb], axis=0)
  w2r, b2r = _rep_w2(gate_W2[0], gate_b2[0])
  all_emb, pk0 = _init_fuse(a, b, gate_W1[0], gate_b1[0:1], w2r, b2r)

  base_edges = _prep_edges(base_edge_index, base_edge_weight)
  cooc_edges = _prep_edges(cooccur_edge_index, cooccur_edge_weight)

  spmm = _make_spmm()
  zrows = jnp.zeros((_SLAB, _H), jnp.float32)
  ones_g = jnp.ones((_N_USERS, _H), jnp.float32)
  xcur = pk0.reshape(_NC * _N, 16)
  acc = all_emb.reshape(_NC * _N, _H)
  for layer in range(1, n_layers + 1):
    ybase = spmm(xcur, zrows, *base_edges)
    ycooc = spmm(xcur, zrows, *cooc_edges)
    w2r, b2r = _rep_w2(gate_W2[layer], gate_b2[layer])
    g_items = _item_gates(ybase.reshape(2, _N, _H), herb_emb,
                          gate_W1[layer], gate_b1[layer:layer + 1], w2r, b2r)
    gfull = jnp.concatenate([ones_g, g_items], axis=0)
    scale = 1.0 / (n_layers + 1) if layer == n_layers else 1.0
    xcur, acc = _make_fuse(scale)(ybase, ycooc, gfull, acc)

  users = jnp.concatenate([acc[:_N_USERS], acc[_N: _N + _N_USERS]], axis=1)
  items = jnp.concatenate([acc[_N_USERS:_N], acc[_N + _N_USERS:]], axis=1)
  return (users, items)
